# baseline (speedup 1.0000x reference)
import functools

import jax
import jax.numpy as jnp
from jax import lax
from jax.experimental import pallas as pl
from jax.experimental.pallas import tpu as pltpu
from jax.experimental.pallas import tpu_sc as plsc

N = 10000
E = 320000
SEQ = 10
NP = 10240
K = 128
NTILES = 32
CHUNKS = 82
EPT = CHUNKS * K
EP = NTILES * EPT
ROWS_PT = NP // 16
NI = CHUNKS // 2

_mesh = plsc.VectorSubcoreMesh(core_axis_name="c", subcore_axis_name="s")
_params = pltpu.CompilerParams(needs_layout_passes=False,
                               use_tc_tiling_on_sc=False)


@functools.partial(
    pl.kernel,
    out_type=jax.ShapeDtypeStruct((SEQ * 2 * NP, 72), jnp.float32),
    mesh=_mesh,
    compiler_params=_params,
    scratch_types=(
        pltpu.VMEM_SHARED((NP, 72), jnp.float32),
        pltpu.VMEM((CHUNKS, 2, K), jnp.int32),
        pltpu.VMEM((2, K), jnp.int32),
        pltpu.VMEM((2, K), jnp.int32),
        pltpu.VMEM((2, K, 8), jnp.float32),
        pltpu.VMEM((2, K, 72), jnp.float32),
        pltpu.VMEM((2, K, 72), jnp.float32),
        pltpu.SemaphoreType.DMA,
        pltpu.SemaphoreType.DMA,
        pltpu.SemaphoreType.DMA,
        pltpu.SemaphoreType.DMA,
    ),
)
def _gat1_sc(h1x, ad1, sdh, z72, out_agg,
             agg_sp, sdv, idxs, idxd, adb, hbuf, mbuf, gA, gB, sA, sB):
    c = lax.axis_index("c")
    s = lax.axis_index("s")
    wid = c * 16 + s
    iota = lax.iota(jnp.int32, 16)
    rvec0 = jnp.where(iota >= 8, 1, 0)
    cvec = iota & 7
    cvec64 = cvec + 64
    colk = [64 + 2 * k + rvec0 for k in range(4)]
    row0 = s * ROWS_PT
    gsem = (gA, gB)
    ssem = (sA, sB)

    pltpu.sync_copy(sdh.at[pl.ds(wid * CHUNKS, CHUNKS)], sdv)

    def zero_acc():
        pltpu.sync_copy(z72.at[pl.ds(row0, ROWS_PT)],
                        agg_sp.at[pl.ds(row0, ROWS_PT)])

    def gather_descs(S):
        return (pltpu.make_async_copy(h1x.at[idxs.at[S]], hbuf.at[S], gsem[S]),
                pltpu.make_async_copy(ad1.at[idxd.at[S]], adb.at[S], gsem[S]))

    def scatter_descs(S, ci):
        return (pltpu.make_async_copy(mbuf.at[S],
                                      agg_sp.at[sdv.at[ci, 1]], ssem[S]),)

    def issue_gathers(ci, S, toff):
        @plsc.parallel_loop(0, K // 16)
        def _(i):
            sl = pl.ds(i * 16, 16)
            idxs[S, sl] = sdv[ci, 0, sl] + toff
            idxd[S, sl] = sdv[ci, 1, sl] + toff
        for d in gather_descs(S):
            d.start()

    def compute(S):
        @plsc.parallel_loop(0, K // 2, unroll=4)
        def _(j):
            rv = rvec0 + 2 * j
            a16 = plsc.load_gather(hbuf.at[S], [rv, cvec64])
            d16 = plsc.load_gather(adb.at[S], [rv, cvec])
            v = a16 + d16
            v = jnp.maximum(v, 0.2 * v)
            plsc.store_scatter(mbuf.at[S], [rv, cvec64], jnp.exp(v))

        @plsc.parallel_loop(0, K, unroll=2)
        def _(e):
            erow = jnp.full((16,), e, jnp.int32)
            for k in range(4):
                w = plsc.load_gather(mbuf.at[S], [erow, colk[k]])
                hv = hbuf[S, e, pl.ds(k * 16, 16)]
                mbuf[S, e, pl.ds(k * 16, 16)] = hv * w

    zero_acc()
    plsc.subcore_barrier()
    for t in range(SEQ):
        toff = t * NP
        issue_gathers(0, 0, toff)
        issue_gathers(1, 1, toff)

        def iter_body(i, _, toff=toff):
            for S in (0, 1):
                ci = 2 * i + S
                for d in gather_descs(S):
                    d.wait()

                @pl.when(i >= 1)
                def _():
                    for d in scatter_descs(S, ci - 2):
                        d.wait()

                compute(S)
                pltpu.async_copy(mbuf.at[S], agg_sp.at[sdv.at[ci, 1]],
                                 ssem[S], add=True)

                @pl.when(i < NI - 1)
                def _():
                    issue_gathers(ci + 2, S, toff)
            return 0

        lax.fori_loop(0, NI, iter_body, 0)
        for S in (0, 1):
            for d in scatter_descs(S, CHUNKS - 2 + S):
                d.wait()
        plsc.subcore_barrier()
        off = (t * 2 + c) * NP + row0
        pltpu.sync_copy(agg_sp.at[pl.ds(row0, ROWS_PT)],
                        out_agg.at[pl.ds(off, ROWS_PT)])
        if t < SEQ - 1:
            zero_acc()
        plsc.subcore_barrier()


@functools.partial(
    pl.kernel,
    out_type=jax.ShapeDtypeStruct((SEQ * 2 * NP, 8), jnp.float32),
    mesh=_mesh,
    compiler_params=_params,
    scratch_types=(
        pltpu.VMEM_SHARED((NP, 8), jnp.float32),
        pltpu.VMEM((CHUNKS, 2, K), jnp.int32),
        pltpu.VMEM((2, K), jnp.int32),
        pltpu.VMEM((2, K), jnp.int32),
        pltpu.VMEM((2, K, 8), jnp.float32),
        pltpu.VMEM((2, K, 8), jnp.float32),
        pltpu.VMEM((2, K, 8), jnp.float32),
        pltpu.SemaphoreType.DMA,
        pltpu.SemaphoreType.DMA,
        pltpu.SemaphoreType.DMA,
        pltpu.SemaphoreType.DMA,
    ),
)
def _gat2_sc(r2, sdh, z8, out_agg,
             agg_sp, sdv, idxs, idxd, sb, db, mb, gA, gB, sA, sB):
    c = lax.axis_index("c")
    s = lax.axis_index("s")
    wid = c * 16 + s
    iota = lax.iota(jnp.int32, 16)
    rvec0 = jnp.where(iota >= 8, 1, 0)
    cvec = iota & 7
    c6 = jnp.full((16,), 6, jnp.int32)
    c7 = jnp.full((16,), 7, jnp.int32)
    row0 = s * ROWS_PT
    gsem = (gA, gB)
    ssem = (sA, sB)

    pltpu.sync_copy(sdh.at[pl.ds(wid * CHUNKS, CHUNKS)], sdv)

    def zero_acc():
        pltpu.sync_copy(z8.at[pl.ds(row0, ROWS_PT)],
                        agg_sp.at[pl.ds(row0, ROWS_PT)])

    def gather_descs(S):
        return (pltpu.make_async_copy(r2.at[idxs.at[S]], sb.at[S], gsem[S]),
                pltpu.make_async_copy(r2.at[idxd.at[S]], db.at[S], gsem[S]))

    def scatter_descs(S, ci):
        return (pltpu.make_async_copy(mb.at[S],
                                      agg_sp.at[sdv.at[ci, 1]], ssem[S]),)

    def issue_gathers(ci, S, toff):
        @plsc.parallel_loop(0, K // 16)
        def _(i):
            sl = pl.ds(i * 16, 16)
            idxs[S, sl] = sdv[ci, 0, sl] + toff
            idxd[S, sl] = sdv[ci, 1, sl] + toff
        for d in gather_descs(S):
            d.start()

    def compute(S):
        @plsc.parallel_loop(0, K // 2, unroll=4)
        def _(j):
            rv = rvec0 + 2 * j
            asg = plsc.load_gather(sb.at[S], [rv, c6])
            adg = plsc.load_gather(db.at[S], [rv, c7])
            v = asg + adg
            v = jnp.maximum(v, 0.2 * v)
            ex = jnp.exp(v)
            m16 = plsc.load_gather(sb.at[S], [rv, cvec])
            plsc.store_scatter(mb.at[S], [rv, cvec], m16 * ex)

    zero_acc()
    plsc.subcore_barrier()
    for t in range(SEQ):
        toff = t * NP
        issue_gathers(0, 0, toff)
        issue_gathers(1, 1, toff)

        def iter_body(i, _, toff=toff):
            for S in (0, 1):
                ci = 2 * i + S
                for d in gather_descs(S):
                    d.wait()

                @pl.when(i >= 1)
                def _():
                    for d in scatter_descs(S, ci - 2):
                        d.wait()

                compute(S)
                pltpu.async_copy(mb.at[S], agg_sp.at[sdv.at[ci, 1]],
                                 ssem[S], add=True)

                @pl.when(i < NI - 1)
                def _():
                    issue_gathers(ci + 2, S, toff)
            return 0

        lax.fori_loop(0, NI, iter_body, 0)
        for S in (0, 1):
            for d in scatter_descs(S, CHUNKS - 2 + S):
                d.wait()
        plsc.subcore_barrier()
        off = (t * 2 + c) * NP + row0
        pltpu.sync_copy(agg_sp.at[pl.ds(row0, ROWS_PT)],
                        out_agg.at[pl.ds(off, ROWS_PT)])
        if t < SEQ - 1:
            zero_acc()
        plsc.subcore_barrier()


_RB = 256
_GA = SEQ * NP // _RB


@functools.partial(
    pl.pallas_call,
    grid=(_GA,),
    in_specs=[
        pl.BlockSpec((_RB, 4), lambda i: (i, 0)),
        pl.BlockSpec((4, 72), lambda i: (0, 0)),
        pl.BlockSpec((4, 8), lambda i: (0, 0)),
    ],
    out_specs=[
        pl.BlockSpec((_RB, 72), lambda i: (i, 0)),
        pl.BlockSpec((_RB, 8), lambda i: (i, 0)),
    ],
    out_shape=[
        jax.ShapeDtypeStruct((SEQ * NP, 72), jnp.float32),
        jax.ShapeDtypeStruct((SEQ * NP, 8), jnp.float32),
    ],
)
def _dense_a(xb, wa, wad, o1, o2):
    xv = xb[...]
    o1[...] = jnp.dot(xv, wa[...], preferred_element_type=jnp.float32)
    o2[...] = jnp.dot(xv, wad[...], preferred_element_type=jnp.float32)


@functools.partial(
    pl.pallas_call,
    grid=(SEQ, NP // _RB),
    in_specs=[
        pl.BlockSpec((1, 1, _RB, 72), lambda t, b: (t, 0, b, 0)),
        pl.BlockSpec((1, 1, _RB, 72), lambda t, b: (t, 1, b, 0)),
        pl.BlockSpec((8, 64), lambda t, b: (0, 0)),
        pl.BlockSpec((1, 64), lambda t, b: (0, 0)),
        pl.BlockSpec((64, 8), lambda t, b: (0, 0)),
        pl.BlockSpec((1, 8), lambda t, b: (0, 0)),
    ],
    out_specs=pl.BlockSpec((1, _RB, 8), lambda t, b: (t, b, 0)),
    out_shape=jax.ShapeDtypeStruct((SEQ, NP, 8), jnp.float32),
)
def _dense_b(p0, p1, rexp, b1r, w2x, cr, o):
    z = p0[0, 0] + p1[0, 0]
    rec = 1.0 / (z[:, 64:72] + 1e-16)
    y = z[:, :64] * jnp.dot(rec, rexp[...],
                            preferred_element_type=jnp.float32) + b1r[...]
    y = jnp.where(y > 0, y, jnp.exp(y) - 1.0)
    o[0] = jnp.dot(y, w2x[...], preferred_element_type=jnp.float32) + cr[...]


@functools.partial(
    pl.pallas_call,
    grid=(SEQ, NP // _RB),
    in_specs=[
        pl.BlockSpec((1, 1, _RB, 8), lambda t, b: (t, 0, b, 0)),
        pl.BlockSpec((1, 1, _RB, 8), lambda t, b: (t, 1, b, 0)),
        pl.BlockSpec((1, 4), lambda t, b: (0, 0)),
    ],
    out_specs=pl.BlockSpec((1, _RB, 4), lambda t, b: (t, b, 0)),
    out_shape=jax.ShapeDtypeStruct((SEQ, NP, 4), jnp.float32),
)
def _dense_c(p0, p1, b2r, o):
    z = p0[0, 0] + p1[0, 0]
    v = z[:, :4] * (1.0 / (z[:, 4:5] + 1e-16)) + b2r[...]
    m = jnp.max(v, axis=-1, keepdims=True)
    d = v - m
    o[0] = d - jnp.log(jnp.sum(jnp.exp(d), axis=-1, keepdims=True))


def kernel(x, edge_index, W1, a_s1, a_d1, b1, W2, a_s2, a_d2, b2):
    loops = jnp.arange(N, dtype=jnp.int32)
    pad = N + jnp.arange(EP - E - N, dtype=jnp.int32) % (NP - N)
    srcp = jnp.concatenate([edge_index[0].astype(jnp.int32), loops, pad])
    dstp = jnp.concatenate([edge_index[1].astype(jnp.int32), loops, pad])
    sdh = jnp.stack([srcp.reshape(NTILES * CHUNKS, K),
                     dstp.reshape(NTILES * CHUNKS, K)], axis=1)
    sdh = (sdh.reshape(CHUNKS, NTILES, 2, K)
           .transpose(1, 0, 2, 3).reshape(NTILES * CHUNKS, 2, K))
    xp = jnp.pad(x, ((0, 0), (0, NP - N), (0, 0)))

    eye8 = jnp.eye(8, dtype=jnp.float32)
    as64 = (a_s1[:, :, None] * eye8[:, None, :]).reshape(64, 8)
    ad64 = (a_d1[:, :, None] * eye8[:, None, :]).reshape(64, 8)
    wa = jnp.concatenate([W1, W1 @ as64], axis=1)
    wad = W1 @ ad64
    w2x = jnp.concatenate(
        [W2, jnp.zeros((64, 2), jnp.float32),
         W2 @ a_s2[0][:, None], W2 @ a_d2[0][:, None]], axis=1)
    cr = jnp.array([[0, 0, 0, 0, 1, 0, 0, 0]], jnp.float32)
    rexp = jnp.repeat(eye8, 8, axis=1)

    z72 = jnp.zeros((NP, 72), jnp.float32)
    z8 = jnp.zeros((NP, 8), jnp.float32)

    h1x, ad1 = _dense_a(xp.reshape(SEQ * NP, 4), wa, wad)

    out1 = _gat1_sc(h1x, ad1, sdh, z72).reshape(SEQ, 2, NP, 72)

    r2 = _dense_b(out1, out1, rexp, b1.reshape(1, 64), w2x, cr)

    agg2 = _gat2_sc(r2.reshape(SEQ * NP, 8), sdh, z8).reshape(SEQ, 2, NP, 8)

    out = _dense_c(agg2, agg2, b2.reshape(1, 4))
    return out[:, :N, :]


# speedup vs baseline: 1.2217x; 1.2217x over previous
import functools

import jax
import jax.numpy as jnp
from jax import lax
from jax.experimental import pallas as pl
from jax.experimental.pallas import tpu as pltpu
from jax.experimental.pallas import tpu_sc as plsc

N = 10000
E = 320000
SEQ = 10
NP = 10240
K = 128
NTILES = 32
CHUNKS = 82
EPT = CHUNKS * K
EP = NTILES * EPT
ROWS_PT = NP // 16
NI = CHUNKS // 2

_mesh = plsc.VectorSubcoreMesh(core_axis_name="c", subcore_axis_name="s")
_params = pltpu.CompilerParams(needs_layout_passes=False,
                               use_tc_tiling_on_sc=False)


@functools.partial(
    pl.kernel,
    out_type=jax.ShapeDtypeStruct((SEQ * 2 * NP, 72), jnp.float32),
    mesh=_mesh,
    compiler_params=_params,
    scratch_types=(
        pltpu.VMEM_SHARED((NP, 72), jnp.float32),
        pltpu.VMEM((CHUNKS, 2, K), jnp.int32),
        pltpu.VMEM((2, K), jnp.int32),
        pltpu.VMEM((2, K), jnp.int32),
        pltpu.VMEM((2, K, 8), jnp.float32),
        pltpu.VMEM((2, K, 72), jnp.float32),
        pltpu.VMEM((2, K, 72), jnp.float32),
        pltpu.SemaphoreType.DMA,
        pltpu.SemaphoreType.DMA,
        pltpu.SemaphoreType.DMA,
        pltpu.SemaphoreType.DMA,
    ),
)
def _gat1_sc(h1x, ad1, sdh, z72, out_agg,
             agg_sp, sdv, idxs, idxd, adb, hbuf, mbuf, gA, gB, sA, sB):
    c = lax.axis_index("c")
    s = lax.axis_index("s")
    wid = c * 16 + s
    iota = lax.iota(jnp.int32, 16)
    rvec0 = jnp.where(iota >= 8, 1, 0)
    cvec = iota & 7
    cvec64 = cvec + 64
    colk = [64 + 2 * k + rvec0 for k in range(4)]
    row0 = s * ROWS_PT
    gsem = (gA, gB)
    ssem = (sA, sB)

    pltpu.sync_copy(sdh.at[pl.ds(wid * CHUNKS, CHUNKS)], sdv)

    def zero_acc():
        pltpu.sync_copy(z72.at[pl.ds(row0, ROWS_PT)],
                        agg_sp.at[pl.ds(row0, ROWS_PT)])

    def gather_descs(S):
        return (pltpu.make_async_copy(h1x.at[idxs.at[S]], hbuf.at[S], gsem[S]),
                pltpu.make_async_copy(ad1.at[idxd.at[S]], adb.at[S], gsem[S]))

    def scatter_descs(S, ci):
        return (pltpu.make_async_copy(mbuf.at[S],
                                      agg_sp.at[sdv.at[ci, 1]], ssem[S]),)

    def issue_gathers(ci, S, toff):
        @plsc.parallel_loop(0, K // 16)
        def _(i):
            sl = pl.ds(i * 16, 16)
            idxs[S, sl] = sdv[ci, 0, sl] + toff
            idxd[S, sl] = sdv[ci, 1, sl] + toff
        for d in gather_descs(S):
            d.start()

    def compute(S):
        @plsc.parallel_loop(0, K // 2, unroll=4)
        def _(j):
            rv = rvec0 + 2 * j
            a16 = plsc.load_gather(hbuf.at[S], [rv, cvec64])
            d16 = plsc.load_gather(adb.at[S], [rv, cvec])
            v = a16 + d16
            v = jnp.maximum(v, 0.2 * v)
            plsc.store_scatter(mbuf.at[S], [rv, cvec64], jnp.exp(v))

        @plsc.parallel_loop(0, K, unroll=2)
        def _(e):
            erow = jnp.full((16,), e, jnp.int32)
            for k in range(4):
                w = plsc.load_gather(mbuf.at[S], [erow, colk[k]])
                hv = hbuf[S, e, pl.ds(k * 16, 16)]
                mbuf[S, e, pl.ds(k * 16, 16)] = hv * w

    zero_acc()
    plsc.subcore_barrier()
    for t in range(SEQ):
        toff = t * NP
        issue_gathers(0, 0, toff)
        issue_gathers(1, 1, toff)

        def iter_body(i, _, toff=toff):
            for S in (0, 1):
                ci = 2 * i + S
                for d in gather_descs(S):
                    d.wait()

                @pl.when(i >= 1)
                def _():
                    for d in scatter_descs(S, ci - 2):
                        d.wait()

                compute(S)
                pltpu.async_copy(mbuf.at[S], agg_sp.at[sdv.at[ci, 1]],
                                 ssem[S], add=True)

                @pl.when(i < NI - 1)
                def _():
                    issue_gathers(ci + 2, S, toff)
            return 0

        lax.fori_loop(0, NI, iter_body, 0)
        for S in (0, 1):
            for d in scatter_descs(S, CHUNKS - 2 + S):
                d.wait()
        plsc.subcore_barrier()
        off = (t * 2 + c) * NP + row0
        pltpu.sync_copy(agg_sp.at[pl.ds(row0, ROWS_PT)],
                        out_agg.at[pl.ds(off, ROWS_PT)])
        if t < SEQ - 1:
            zero_acc()
        plsc.subcore_barrier()


@functools.partial(
    pl.kernel,
    out_type=jax.ShapeDtypeStruct((SEQ * 2 * NP, 8), jnp.float32),
    mesh=_mesh,
    compiler_params=_params,
    scratch_types=(
        pltpu.VMEM_SHARED((NP, 8), jnp.float32),
        pltpu.VMEM((CHUNKS, 2, K), jnp.int32),
        pltpu.VMEM((2, K), jnp.int32),
        pltpu.VMEM((2, K), jnp.int32),
        pltpu.VMEM((2, K, 8), jnp.float32),
        pltpu.VMEM((2, K, 8), jnp.float32),
        pltpu.VMEM((2, K, 8), jnp.float32),
        pltpu.SemaphoreType.DMA,
        pltpu.SemaphoreType.DMA,
        pltpu.SemaphoreType.DMA,
        pltpu.SemaphoreType.DMA,
    ),
)
def _gat2_sc(r2, sdh, z8, out_agg,
             agg_sp, sdv, idxs, idxd, sb, db, mb, gA, gB, sA, sB):
    c = lax.axis_index("c")
    s = lax.axis_index("s")
    wid = c * 16 + s
    iota = lax.iota(jnp.int32, 16)
    rvec0 = jnp.where(iota >= 8, 1, 0)
    cvec = iota & 7
    c6 = jnp.full((16,), 6, jnp.int32)
    c7 = jnp.full((16,), 7, jnp.int32)
    row0 = s * ROWS_PT
    gsem = (gA, gB)
    ssem = (sA, sB)

    pltpu.sync_copy(sdh.at[pl.ds(wid * CHUNKS, CHUNKS)], sdv)

    def zero_acc():
        pltpu.sync_copy(z8.at[pl.ds(row0, ROWS_PT)],
                        agg_sp.at[pl.ds(row0, ROWS_PT)])

    def gather_descs(S):
        return (pltpu.make_async_copy(r2.at[idxs.at[S]], sb.at[S], gsem[S]),
                pltpu.make_async_copy(r2.at[idxd.at[S]], db.at[S], gsem[S]))

    def scatter_descs(S, ci):
        return (pltpu.make_async_copy(mb.at[S],
                                      agg_sp.at[sdv.at[ci, 1]], ssem[S]),)

    def issue_gathers(ci, S, toff):
        @plsc.parallel_loop(0, K // 16)
        def _(i):
            sl = pl.ds(i * 16, 16)
            idxs[S, sl] = sdv[ci, 0, sl] + toff
            idxd[S, sl] = sdv[ci, 1, sl] + toff
        for d in gather_descs(S):
            d.start()

    def compute(S):
        @plsc.parallel_loop(0, K // 2, unroll=4)
        def _(j):
            rv = rvec0 + 2 * j
            asg = plsc.load_gather(sb.at[S], [rv, c6])
            adg = plsc.load_gather(db.at[S], [rv, c7])
            v = asg + adg
            v = jnp.maximum(v, 0.2 * v)
            ex = jnp.exp(v)
            m16 = plsc.load_gather(sb.at[S], [rv, cvec])
            plsc.store_scatter(mb.at[S], [rv, cvec], m16 * ex)

    zero_acc()
    plsc.subcore_barrier()
    for t in range(SEQ):
        toff = t * NP
        issue_gathers(0, 0, toff)
        issue_gathers(1, 1, toff)

        def iter_body(i, _, toff=toff):
            for S in (0, 1):
                ci = 2 * i + S
                for d in gather_descs(S):
                    d.wait()

                @pl.when(i >= 1)
                def _():
                    for d in scatter_descs(S, ci - 2):
                        d.wait()

                compute(S)
                pltpu.async_copy(mb.at[S], agg_sp.at[sdv.at[ci, 1]],
                                 ssem[S], add=True)

                @pl.when(i < NI - 1)
                def _():
                    issue_gathers(ci + 2, S, toff)
            return 0

        lax.fori_loop(0, NI, iter_body, 0)
        for S in (0, 1):
            for d in scatter_descs(S, CHUNKS - 2 + S):
                d.wait()
        plsc.subcore_barrier()
        off = (t * 2 + c) * NP + row0
        pltpu.sync_copy(agg_sp.at[pl.ds(row0, ROWS_PT)],
                        out_agg.at[pl.ds(off, ROWS_PT)])
        if t < SEQ - 1:
            zero_acc()
        plsc.subcore_barrier()


_RB = 1024
_GA = SEQ * NP // _RB


@functools.partial(
    pl.pallas_call,
    grid=(_GA,),
    in_specs=[
        pl.BlockSpec((_RB, 4), lambda i: (i, 0)),
        pl.BlockSpec((4, 72), lambda i: (0, 0)),
        pl.BlockSpec((4, 8), lambda i: (0, 0)),
    ],
    out_specs=[
        pl.BlockSpec((_RB, 72), lambda i: (i, 0)),
        pl.BlockSpec((_RB, 8), lambda i: (i, 0)),
    ],
    out_shape=[
        jax.ShapeDtypeStruct((SEQ * NP, 72), jnp.float32),
        jax.ShapeDtypeStruct((SEQ * NP, 8), jnp.float32),
    ],
)
def _dense_a(xb, wa, wad, o1, o2):
    xv = xb[...]
    o1[...] = jnp.dot(xv, wa[...], preferred_element_type=jnp.float32)
    o2[...] = jnp.dot(xv, wad[...], preferred_element_type=jnp.float32)


@functools.partial(
    pl.pallas_call,
    grid=(SEQ, NP // _RB),
    in_specs=[
        pl.BlockSpec((1, 2, _RB, 72), lambda t, b: (t, 0, b, 0)),
        pl.BlockSpec((8, 64), lambda t, b: (0, 0)),
        pl.BlockSpec((1, 64), lambda t, b: (0, 0)),
        pl.BlockSpec((64, 8), lambda t, b: (0, 0)),
        pl.BlockSpec((1, 8), lambda t, b: (0, 0)),
    ],
    out_specs=pl.BlockSpec((1, _RB, 8), lambda t, b: (t, b, 0)),
    out_shape=jax.ShapeDtypeStruct((SEQ, NP, 8), jnp.float32),
)
def _dense_b(p, rexp, b1r, w2x, cr, o):
    z = p[0, 0] + p[0, 1]
    rec = 1.0 / (z[:, 64:72] + 1e-16)
    y = z[:, :64] * jnp.dot(rec, rexp[...],
                            preferred_element_type=jnp.float32) + b1r[...]
    y = jnp.where(y > 0, y, jnp.exp(y) - 1.0)
    o[0] = jnp.dot(y, w2x[...], preferred_element_type=jnp.float32) + cr[...]


@functools.partial(
    pl.pallas_call,
    grid=(SEQ, NP // _RB),
    in_specs=[
        pl.BlockSpec((1, 2, _RB, 8), lambda t, b: (t, 0, b, 0)),
        pl.BlockSpec((1, 4), lambda t, b: (0, 0)),
    ],
    out_specs=pl.BlockSpec((1, _RB, 4), lambda t, b: (t, b, 0)),
    out_shape=jax.ShapeDtypeStruct((SEQ, NP, 4), jnp.float32),
)
def _dense_c(p, b2r, o):
    z = p[0, 0] + p[0, 1]
    v = z[:, :4] * (1.0 / (z[:, 4:5] + 1e-16)) + b2r[...]
    m = jnp.max(v, axis=-1, keepdims=True)
    d = v - m
    o[0] = d - jnp.log(jnp.sum(jnp.exp(d), axis=-1, keepdims=True))


def kernel(x, edge_index, W1, a_s1, a_d1, b1, W2, a_s2, a_d2, b2):
    loops = jnp.arange(N, dtype=jnp.int32)
    pad = N + jnp.arange(EP - E - N, dtype=jnp.int32) % (NP - N)
    srcp = jnp.concatenate([edge_index[0].astype(jnp.int32), loops, pad])
    dstp = jnp.concatenate([edge_index[1].astype(jnp.int32), loops, pad])
    sdh = jnp.stack([srcp.reshape(NTILES * CHUNKS, K),
                     dstp.reshape(NTILES * CHUNKS, K)], axis=1)
    sdh = (sdh.reshape(CHUNKS, NTILES, 2, K)
           .transpose(1, 0, 2, 3).reshape(NTILES * CHUNKS, 2, K))
    xp = jnp.pad(x, ((0, 0), (0, NP - N), (0, 0)))

    eye8 = jnp.eye(8, dtype=jnp.float32)
    as64 = (a_s1[:, :, None] * eye8[:, None, :]).reshape(64, 8)
    ad64 = (a_d1[:, :, None] * eye8[:, None, :]).reshape(64, 8)
    wa = jnp.concatenate([W1, W1 @ as64], axis=1)
    wad = W1 @ ad64
    w2x = jnp.concatenate(
        [W2, jnp.zeros((64, 2), jnp.float32),
         W2 @ a_s2[0][:, None], W2 @ a_d2[0][:, None]], axis=1)
    cr = jnp.array([[0, 0, 0, 0, 1, 0, 0, 0]], jnp.float32)
    rexp = jnp.repeat(eye8, 8, axis=1)

    z72 = jnp.zeros((NP, 72), jnp.float32)
    z8 = jnp.zeros((NP, 8), jnp.float32)

    h1x, ad1 = _dense_a(xp.reshape(SEQ * NP, 4), wa, wad)

    out1 = _gat1_sc(h1x, ad1, sdh, z72).reshape(SEQ, 2, NP, 72)

    r2 = _dense_b(out1, rexp, b1.reshape(1, 64), w2x, cr)

    agg2 = _gat2_sc(r2.reshape(SEQ * NP, 8), sdh, z8).reshape(SEQ, 2, NP, 8)

    out = _dense_c(agg2, b2.reshape(1, 4))
    return out[:, :N, :]
